# baseline (speedup 1.0000x reference)
import functools

import jax
import jax.numpy as jnp
from jax import lax
from jax.experimental import pallas as pl
from jax.experimental.pallas import tpu as pltpu
from jax.experimental.pallas import tpu_sc as plsc

_NW = 32
_LANES = 16
_PADW = 128


def _pooled_lookup(items_p, table_p, B, S, D, SP):
    bpw = B // _NW
    nlg = D // _LANES
    n1 = min(S, 128)
    n2 = S - n1
    mesh = plsc.VectorSubcoreMesh(core_axis_name="c", subcore_axis_name="s")

    @functools.partial(
        pl.kernel,
        out_type=jax.ShapeDtypeStruct((B, _PADW), jnp.float32),
        mesh=mesh,
        scratch_types=[
            pltpu.VMEM((bpw, SP), jnp.int32),
            pltpu.VMEM((S, _PADW), jnp.float32),
            pltpu.VMEM((S, _PADW), jnp.float32),
            pltpu.VMEM((bpw, _PADW), jnp.float32),
            pltpu.SemaphoreType.DMA,
            pltpu.SemaphoreType.DMA,
        ],
        compiler_params=pltpu.CompilerParams(use_tc_tiling_on_sc=True),
    )
    def k(table_hbm, items_hbm, out_hbm, idx_v, buf0, buf1, out_v, sem0, sem1):
        wid = lax.axis_index("s") * 2 + lax.axis_index("c")
        base = wid * bpw
        pltpu.sync_copy(items_hbm.at[pl.ds(base, bpw)], idx_v)

        inv = jnp.float32(1.0 / S)

        def gather_row(b, buf, sem):
            pltpu.async_copy(
                table_hbm.at[idx_v.at[b, pl.ds(0, n1)]], buf.at[pl.ds(0, n1)], sem
            )
            pltpu.async_copy(
                table_hbm.at[idx_v.at[b, pl.ds(n1, n2)]], buf.at[pl.ds(n1, n2)], sem
            )

        def wait_row(b, buf, sem):
            pltpu.make_async_copy(
                table_hbm.at[idx_v.at[b, pl.ds(0, n1)]], buf.at[pl.ds(0, n1)], sem
            ).wait()
            pltpu.make_async_copy(
                table_hbm.at[idx_v.at[b, pl.ds(n1, n2)]], buf.at[pl.ds(n1, n2)], sem
            ).wait()

        unroll = 8
        assert S % unroll == 0

        def accumulate(buf, b):
            def body(r0, accs):
                r = r0 * unroll
                for u in range(unroll):
                    accs = tuple(
                        accs[g] + buf[r + u, pl.ds(_LANES * g, _LANES)]
                        for g in range(nlg)
                    )
                return accs

            accs = tuple(jnp.zeros((_LANES,), jnp.float32) for _ in range(nlg))
            accs = lax.fori_loop(0, S // unroll, body, accs)
            for g in range(nlg):
                out_v[b, pl.ds(_LANES * g, _LANES)] = accs[g] * inv

        gather_row(0, buf0, sem0)

        @pl.loop(0, bpw, step=2)
        def _(b):
            gather_row(b + 1, buf1, sem1)
            wait_row(b, buf0, sem0)
            accumulate(buf0, b)

            @pl.when(b + 2 < bpw)
            def _():
                gather_row(b + 2, buf0, sem0)

            wait_row(b + 1, buf1, sem1)
            accumulate(buf1, b + 1)

        pltpu.sync_copy(out_v, out_hbm.at[pl.ds(base, bpw)])

    return k(table_p, items_p)


_RB = 4096


def _transpose_pad(tableT, V, D):
    grid = (V + _RB - 1) // _RB

    def body(x_ref, o_ref):
        o_ref[:, 0:D] = jnp.swapaxes(x_ref[...], 0, 1)

    return pl.pallas_call(
        body,
        grid=(grid,),
        in_specs=[pl.BlockSpec((D, _RB), lambda i: (0, i))],
        out_specs=pl.BlockSpec((_RB, _PADW), lambda i: (i, 0)),
        out_shape=jax.ShapeDtypeStruct((V, _PADW), jnp.float32),
    )(tableT)


def kernel(items, table):
    B, S = items.shape
    V, D = table.shape
    SP = 256
    items_p = jnp.pad(items.astype(jnp.int32), ((0, 0), (0, SP - S)))
    table_p = _transpose_pad(table.T, V, D)
    out = _pooled_lookup(items_p, table_p, B, S, D, SP)
    return out[:, :D]


# speedup vs baseline: 1.0112x; 1.0112x over previous
import functools

import jax
import jax.numpy as jnp
from jax import lax
from jax.experimental import pallas as pl
from jax.experimental.pallas import tpu as pltpu
from jax.experimental.pallas import tpu_sc as plsc

_NW = 32
_LANES = 16
_PADW = 128
_NBUF = 2


def _pooled_lookup(items_p, table_p, B, S, D, SP):
    bpw = B // _NW
    nlg = D // _LANES
    n1 = min(S, 128)
    n2 = S - n1
    assert bpw % _NBUF == 0
    mesh = plsc.VectorSubcoreMesh(core_axis_name="c", subcore_axis_name="s")

    @functools.partial(
        pl.kernel,
        out_type=jax.ShapeDtypeStruct((B, _PADW), jnp.float32),
        mesh=mesh,
        scratch_types=[
            pltpu.VMEM((bpw, SP), jnp.int32),
            *([pltpu.VMEM((S, _PADW), jnp.float32)] * _NBUF),
            pltpu.VMEM((bpw, _PADW), jnp.float32),
            *([pltpu.SemaphoreType.DMA] * _NBUF),
        ],
        compiler_params=pltpu.CompilerParams(use_tc_tiling_on_sc=True),
    )
    def k(table_hbm, items_hbm, out_hbm, idx_v, *rest):
        bufs = rest[:_NBUF]
        out_v = rest[_NBUF]
        sems = rest[_NBUF + 1 : 2 * _NBUF + 1]
        wid = lax.axis_index("s") * 2 + lax.axis_index("c")
        base = wid * bpw
        pltpu.sync_copy(items_hbm.at[pl.ds(base, bpw)], idx_v)

        inv = jnp.float32(1.0 / S)

        def gather_row(b, buf, sem):
            pltpu.async_copy(
                table_hbm.at[idx_v.at[b, pl.ds(0, n1)]], buf.at[pl.ds(0, n1)], sem
            )
            pltpu.async_copy(
                table_hbm.at[idx_v.at[b, pl.ds(n1, n2)]], buf.at[pl.ds(n1, n2)], sem
            )

        def wait_row(b, buf, sem):
            pltpu.make_async_copy(
                table_hbm.at[idx_v.at[b, pl.ds(0, n1)]], buf.at[pl.ds(0, n1)], sem
            ).wait()
            pltpu.make_async_copy(
                table_hbm.at[idx_v.at[b, pl.ds(n1, n2)]], buf.at[pl.ds(n1, n2)], sem
            ).wait()

        def accumulate(buf, b):
            def body(r, accs):
                return tuple(
                    accs[g] + buf[r, pl.ds(_LANES * g, _LANES)] for g in range(nlg)
                )

            accs = tuple(jnp.zeros((_LANES,), jnp.float32) for _ in range(nlg))
            accs = lax.fori_loop(0, S, body, accs)
            for g in range(nlg):
                out_v[b, pl.ds(_LANES * g, _LANES)] = accs[g] * inv

        for u in range(_NBUF - 1):
            gather_row(u, bufs[u], sems[u])

        @pl.loop(0, bpw, step=_NBUF)
        def _(b):
            for u in range(_NBUF):
                bn = b + u + _NBUF - 1

                @pl.when(bn < bpw)
                def _():
                    gather_row(
                        bn, bufs[(u + _NBUF - 1) % _NBUF], sems[(u + _NBUF - 1) % _NBUF]
                    )

                wait_row(b + u, bufs[u], sems[u])
                accumulate(bufs[u], b + u)

        pltpu.sync_copy(out_v, out_hbm.at[pl.ds(base, bpw)])

    return k(table_p, items_p)


_RB = 4096


def _transpose_pad(tableT, V, D):
    grid = (V + _RB - 1) // _RB

    def body(x_ref, o_ref):
        o_ref[:, 0:D] = jnp.swapaxes(x_ref[...], 0, 1)

    return pl.pallas_call(
        body,
        grid=(grid,),
        in_specs=[pl.BlockSpec((D, _RB), lambda i: (0, i))],
        out_specs=pl.BlockSpec((_RB, _PADW), lambda i: (i, 0)),
        out_shape=jax.ShapeDtypeStruct((V, _PADW), jnp.float32),
    )(tableT)


def kernel(items, table):
    B, S = items.shape
    V, D = table.shape
    SP = 256
    items_p = jnp.pad(items.astype(jnp.int32), ((0, 0), (0, SP - S)))
    table_p = _transpose_pad(table.T, V, D)
    out = _pooled_lookup(items_p, table_p, B, S, D, SP)
    return out[:, :D]


# speedup vs baseline: 1.1621x; 1.1492x over previous
import functools

import jax
import jax.numpy as jnp
from jax import lax
from jax.experimental import pallas as pl
from jax.experimental.pallas import tpu as pltpu
from jax.experimental.pallas import tpu_sc as plsc

_NW = 32
_LANES = 16
_PADW = 128
_NBUF = 2


def _pooled_lookup(items_p, table_p, B, S, D, SP):
    bpw = B // _NW
    nlg = D // _LANES
    n1 = min(S, 128)
    n2 = S - n1
    assert bpw % _NBUF == 0
    mesh = plsc.VectorSubcoreMesh(core_axis_name="c", subcore_axis_name="s")

    @functools.partial(
        pl.kernel,
        out_type=jax.ShapeDtypeStruct((B, _PADW), jnp.float32),
        mesh=mesh,
        scratch_types=[
            pltpu.VMEM((bpw, SP), jnp.int32),
            *([pltpu.VMEM((S, _PADW), jnp.float32)] * _NBUF),
            pltpu.VMEM((bpw, _PADW), jnp.float32),
            *([pltpu.SemaphoreType.DMA] * _NBUF),
        ],
        compiler_params=pltpu.CompilerParams(use_tc_tiling_on_sc=True),
    )
    def k(table_hbm, items_hbm, out_hbm, idx_v, *rest):
        bufs = rest[:_NBUF]
        out_v = rest[_NBUF]
        sems = rest[_NBUF + 1 : 2 * _NBUF + 1]
        wid = lax.axis_index("s") * 2 + lax.axis_index("c")
        base = wid * bpw
        pltpu.sync_copy(items_hbm.at[pl.ds(base, bpw)], idx_v)

        inv = jnp.float32(1.0 / S)

        def gather_row(b, buf, sem):
            pltpu.async_copy(
                table_hbm.at[idx_v.at[b, pl.ds(0, n1)]], buf.at[pl.ds(0, n1)], sem
            )
            pltpu.async_copy(
                table_hbm.at[idx_v.at[b, pl.ds(n1, n2)]], buf.at[pl.ds(n1, n2)], sem
            )

        def wait_row(b, buf, sem):
            pltpu.make_async_copy(
                table_hbm.at[idx_v.at[b, pl.ds(0, n1)]], buf.at[pl.ds(0, n1)], sem
            ).wait()
            pltpu.make_async_copy(
                table_hbm.at[idx_v.at[b, pl.ds(n1, n2)]], buf.at[pl.ds(n1, n2)], sem
            ).wait()

        def accumulate(buf, b):
            def body(r, accs):
                return tuple(
                    accs[g] + buf[r, pl.ds(_LANES * g, _LANES)] for g in range(nlg)
                )

            accs = tuple(jnp.zeros((_LANES,), jnp.float32) for _ in range(nlg))
            accs = lax.fori_loop(0, S, body, accs)
            for g in range(nlg):
                out_v[b, pl.ds(_LANES * g, _LANES)] = accs[g] * inv

        for u in range(_NBUF - 1):
            gather_row(u, bufs[u], sems[u])

        @pl.loop(0, bpw, step=_NBUF)
        def _(b):
            for u in range(_NBUF):
                bn = b + u + _NBUF - 1

                @pl.when(bn < bpw)
                def _():
                    gather_row(
                        bn, bufs[(u + _NBUF - 1) % _NBUF], sems[(u + _NBUF - 1) % _NBUF]
                    )

                wait_row(b + u, bufs[u], sems[u])
                accumulate(bufs[u], b + u)

        pltpu.sync_copy(out_v, out_hbm.at[pl.ds(base, bpw)])

    return k(table_p, items_p)


_RB = 8192


def _transpose_pad(tableT, V, D):
    grid = (V + _RB - 1) // _RB

    def body(x_ref, o_ref):
        o_ref[:, 0:D] = jnp.swapaxes(x_ref[...], 0, 1)

    return pl.pallas_call(
        body,
        grid=(grid,),
        in_specs=[pl.BlockSpec((D, _RB), lambda i: (0, i))],
        out_specs=pl.BlockSpec((_RB, _PADW), lambda i: (i, 0)),
        out_shape=jax.ShapeDtypeStruct((V, _PADW), jnp.float32),
    )(tableT)


def kernel(items, table):
    B, S = items.shape
    V, D = table.shape
    SP = 256
    items_p = jnp.pad(items.astype(jnp.int32), ((0, 0), (0, SP - S)))
    table_p = _transpose_pad(table.T, V, D)
    out = _pooled_lookup(items_p, table_p, B, S, D, SP)
    return out[:, :D]


# speedup vs baseline: 1.2102x; 1.0414x over previous
import functools

import jax
import jax.numpy as jnp
from jax import lax
from jax.experimental import pallas as pl
from jax.experimental.pallas import tpu as pltpu
from jax.experimental.pallas import tpu_sc as plsc

_NW = 32
_LANES = 16
_PADW = 128
_NBUF = 2


def _pooled_lookup(items_p, table_p, B, S, D, SP):
    bpw = B // _NW
    nlg = D // _LANES
    n1 = min(S, 128)
    n2 = S - n1
    assert bpw % _NBUF == 0
    mesh = plsc.VectorSubcoreMesh(core_axis_name="c", subcore_axis_name="s")

    @functools.partial(
        pl.kernel,
        out_type=jax.ShapeDtypeStruct((B, _PADW), jnp.float32),
        mesh=mesh,
        scratch_types=[
            pltpu.VMEM((bpw, SP), jnp.int32),
            *([pltpu.VMEM((S, _PADW), jnp.float32)] * _NBUF),
            pltpu.VMEM((bpw, _PADW), jnp.float32),
            *([pltpu.SemaphoreType.DMA] * _NBUF),
        ],
        compiler_params=pltpu.CompilerParams(use_tc_tiling_on_sc=True),
    )
    def k(table_hbm, items_hbm, out_hbm, idx_v, *rest):
        bufs = rest[:_NBUF]
        out_v = rest[_NBUF]
        sems = rest[_NBUF + 1 : 2 * _NBUF + 1]
        wid = lax.axis_index("s") * 2 + lax.axis_index("c")
        base = wid * bpw
        pltpu.sync_copy(items_hbm.at[pl.ds(base, bpw)], idx_v)

        inv = jnp.float32(1.0 / S)

        def gather_row(b, buf, sem):
            pltpu.async_copy(
                table_hbm.at[idx_v.at[b, pl.ds(0, n1)]], buf.at[pl.ds(0, n1)], sem
            )
            pltpu.async_copy(
                table_hbm.at[idx_v.at[b, pl.ds(n1, n2)]], buf.at[pl.ds(n1, n2)], sem
            )

        def wait_row(b, buf, sem):
            pltpu.make_async_copy(
                table_hbm.at[idx_v.at[b, pl.ds(0, n1)]], buf.at[pl.ds(0, n1)], sem
            ).wait()
            pltpu.make_async_copy(
                table_hbm.at[idx_v.at[b, pl.ds(n1, n2)]], buf.at[pl.ds(n1, n2)], sem
            ).wait()

        def accumulate(buf, b):
            def body(r, accs):
                return tuple(
                    accs[g] + buf[r, pl.ds(_LANES * g, _LANES)] for g in range(nlg)
                )

            accs = tuple(jnp.zeros((_LANES,), jnp.float32) for _ in range(nlg))
            accs = lax.fori_loop(0, S, body, accs)
            for g in range(nlg):
                out_v[b, pl.ds(_LANES * g, _LANES)] = accs[g] * inv

        for u in range(_NBUF - 1):
            gather_row(u, bufs[u], sems[u])

        @pl.loop(0, bpw, step=_NBUF)
        def _(b):
            for u in range(_NBUF):
                bn = b + u + _NBUF - 1

                @pl.when(bn < bpw)
                def _():
                    gather_row(
                        bn, bufs[(u + _NBUF - 1) % _NBUF], sems[(u + _NBUF - 1) % _NBUF]
                    )

                wait_row(b + u, bufs[u], sems[u])
                accumulate(bufs[u], b + u)

        pltpu.sync_copy(out_v, out_hbm.at[pl.ds(base, bpw)])

    return k(table_p, items_p)


_RB = 16384


def _transpose_pad(tableT, V, D):
    grid = (V + _RB - 1) // _RB

    def body(x_ref, o_ref):
        o_ref[:, 0:D] = jnp.swapaxes(x_ref[...], 0, 1)

    return pl.pallas_call(
        body,
        grid=(grid,),
        in_specs=[pl.BlockSpec((D, _RB), lambda i: (0, i))],
        out_specs=pl.BlockSpec((_RB, _PADW), lambda i: (i, 0)),
        out_shape=jax.ShapeDtypeStruct((V, _PADW), jnp.float32),
    )(tableT)


def kernel(items, table):
    B, S = items.shape
    V, D = table.shape
    SP = 256
    items_p = jnp.pad(items.astype(jnp.int32), ((0, 0), (0, SP - S)))
    table_p = _transpose_pad(table.T, V, D)
    out = _pooled_lookup(items_p, table_p, B, S, D, SP)
    return out[:, :D]


# speedup vs baseline: 1.2253x; 1.0125x over previous
import functools

import jax
import jax.numpy as jnp
from jax import lax
from jax.experimental import pallas as pl
from jax.experimental.pallas import tpu as pltpu
from jax.experimental.pallas import tpu_sc as plsc

_NW = 32
_LANES = 16
_PADW = 128
_NBUF = 2


def _pooled_lookup(items_p, table_p, B, S, D, SP):
    bpw = B // _NW
    nlg = D // _LANES
    n1 = min(S, 128)
    n2 = S - n1
    assert bpw % _NBUF == 0
    mesh = plsc.VectorSubcoreMesh(core_axis_name="c", subcore_axis_name="s")

    @functools.partial(
        pl.kernel,
        out_type=jax.ShapeDtypeStruct((B, _PADW), jnp.float32),
        mesh=mesh,
        scratch_types=[
            pltpu.VMEM((bpw, SP), jnp.int32),
            *([pltpu.VMEM((S, _PADW), jnp.float32)] * _NBUF),
            pltpu.VMEM((bpw, _PADW), jnp.float32),
            *([pltpu.SemaphoreType.DMA] * _NBUF),
        ],
        compiler_params=pltpu.CompilerParams(use_tc_tiling_on_sc=True),
    )
    def k(table_hbm, items_hbm, out_hbm, idx_v, *rest):
        bufs = rest[:_NBUF]
        out_v = rest[_NBUF]
        sems = rest[_NBUF + 1 : 2 * _NBUF + 1]
        wid = lax.axis_index("s") * 2 + lax.axis_index("c")
        base = wid * bpw
        pltpu.sync_copy(items_hbm.at[pl.ds(base, bpw)], idx_v)

        inv = jnp.float32(1.0 / S)

        def gather_row(b, buf, sem):
            pltpu.async_copy(
                table_hbm.at[idx_v.at[b, pl.ds(0, n1)]], buf.at[pl.ds(0, n1)], sem
            )
            pltpu.async_copy(
                table_hbm.at[idx_v.at[b, pl.ds(n1, n2)]], buf.at[pl.ds(n1, n2)], sem
            )

        def wait_row(b, buf, sem):
            pltpu.make_async_copy(
                table_hbm.at[idx_v.at[b, pl.ds(0, n1)]], buf.at[pl.ds(0, n1)], sem
            ).wait()
            pltpu.make_async_copy(
                table_hbm.at[idx_v.at[b, pl.ds(n1, n2)]], buf.at[pl.ds(n1, n2)], sem
            ).wait()

        def accumulate(buf, b):
            def body(r, accs):
                return tuple(
                    accs[g] + buf[r, pl.ds(_LANES * g, _LANES)] for g in range(nlg)
                )

            accs = tuple(jnp.zeros((_LANES,), jnp.float32) for _ in range(nlg))
            accs = lax.fori_loop(0, S, body, accs)
            for g in range(nlg):
                out_v[b, pl.ds(_LANES * g, _LANES)] = accs[g] * inv

        for u in range(_NBUF - 1):
            gather_row(u, bufs[u], sems[u])

        @pl.loop(0, bpw, step=_NBUF)
        def _(b):
            for u in range(_NBUF):
                bn = b + u + _NBUF - 1

                @pl.when(bn < bpw)
                def _():
                    gather_row(
                        bn, bufs[(u + _NBUF - 1) % _NBUF], sems[(u + _NBUF - 1) % _NBUF]
                    )

                wait_row(b + u, bufs[u], sems[u])
                accumulate(bufs[u], b + u)

        pltpu.sync_copy(out_v, out_hbm.at[pl.ds(base, bpw)])

    return k(table_p, items_p)


_RB = 32768


def _transpose_pad(tableT, V, D):
    grid = (V + _RB - 1) // _RB

    def body(x_ref, o_ref):
        o_ref[:, 0:D] = jnp.swapaxes(x_ref[...], 0, 1)

    return pl.pallas_call(
        body,
        grid=(grid,),
        in_specs=[pl.BlockSpec((D, _RB), lambda i: (0, i))],
        out_specs=pl.BlockSpec((_RB, _PADW), lambda i: (i, 0)),
        out_shape=jax.ShapeDtypeStruct((V, _PADW), jnp.float32),
    )(tableT)


def kernel(items, table):
    B, S = items.shape
    V, D = table.shape
    SP = 256
    items_p = jnp.pad(items.astype(jnp.int32), ((0, 0), (0, SP - S)))
    table_p = _transpose_pad(table.T, V, D)
    out = _pooled_lookup(items_p, table_p, B, S, D, SP)
    return out[:, :D]
